# Initial kernel scaffold; baseline (speedup 1.0000x reference)
import jax
import jax.numpy as jnp
from jax.experimental import pallas as pl


def kernel(feats, edge_index, edge_type, edge_norm, w_comp1, bases1, bias1, w_comp2, bases2, bias2):
    raise NotImplementedError("write your pallas kernel here")


# speedup vs baseline: 1.8834x; 1.8834x over previous
import functools

import jax
import jax.numpy as jnp
from jax import lax
from jax.experimental import pallas as pl
from jax.experimental.pallas import tpu as pltpu
from jax.experimental.pallas import tpu_sc as plsc

N = 10000
E = 320000
D = 128
R = 16
NB = 8

NPAD = 10240
BN = 2048
NBLK = NPAD // BN
NC = 2
NS = 16
HR = NPAD // NC
NPASS = 2
QR = HR // NPASS
ACC_R = QR + 8
EW = E // NS
EWA = 20096
EWP = EWA + 144
C = 128
ORT = QR // NS
FB = 80
GMASK = (1 << 18) - 1


def _table_body(w_ref, b_ref, x_ref, o_ref):
    r = pl.program_id(0)
    w = jnp.zeros((D, D), jnp.float32)
    for b in range(NB):
        w = w + w_ref[r, b] * b_ref[b]
    o_ref[...] = jnp.dot(x_ref[...], w, preferred_element_type=jnp.float32)


def _tc_table(x, w_comp, bases):
    return pl.pallas_call(
        _table_body,
        grid=(R, NBLK),
        in_specs=[
            pl.BlockSpec(memory_space=pltpu.SMEM),
            pl.BlockSpec((NB, D, D), lambda r, n: (0, 0, 0)),
            pl.BlockSpec((BN, D), lambda r, n: (n, 0)),
        ],
        out_specs=pl.BlockSpec((BN, D), lambda r, n: (r * NBLK + n, 0)),
        out_shape=jax.ShapeDtypeStruct((R * NPAD, D), jnp.float32),
    )(w_comp, bases, x)


def _sc_edge_body(table, pkdr, nrmr, bias, out,
                  pkd_v, dst_v, nrm_v, dstc2, rows, bias_v, acc, sem):
    cid = lax.axis_index("c")
    sid = lax.axis_index("s")

    def init_rows(j, _):
        for k in range(D // 16):
            rows[j, pl.ds(k * 16, 16)] = jnp.zeros((16,), jnp.float32)
        return 0

    lax.fori_loop(0, C, init_rows, 0)
    pltpu.sync_copy(bias, bias_v)

    body = _sc_pass_body
    for p in range(NPASS):
        body(p, cid, sid, table, pkdr, nrmr, out,
             pkd_v, dst_v, nrm_v, dstc2, rows, bias_v, acc)


def _sc_pass_body(p, cid, sid, table, pkdr, nrmr, out,
                  pkd_v, dst_v, nrm_v, dstc2, rows, bias_v, acc):
    base = cid * HR + p * QR

    def init_rows(j, _):
        for k in range(D // 16):
            rows[j, pl.ds(k * 16, 16)] = jnp.zeros((16,), jnp.float32)
        return 0

    lax.fori_loop(0, FB, init_rows, 0)

    def zero_acc(i, _):
        pltpu.sync_copy(rows.at[pl.ds(0, FB)],
                        acc.at[pl.ds(sid * ORT + i * FB, FB)])
        return 0

    lax.fori_loop(0, ORT // FB, zero_acc, 0)

    @pl.when(sid == 0)
    def _():
        pltpu.sync_copy(rows.at[pl.ds(0, ACC_R - QR)],
                        acc.at[pl.ds(QR, ACC_R - QR)])

    pltpu.sync_copy(pkdr.at[sid], pkd_v.at[pl.ds(0, EWA)])
    pltpu.sync_copy(nrmr.at[sid], nrm_v.at[pl.ds(0, EWA)])

    def filt(i, ptr):
        sl = pl.ds(i * 16, 16)
        v = pkd_v[sl]
        g = v & GMASK
        local = lax.shift_right_logical(v, 18) - base
        nv = nrm_v[sl]
        mask = (local >= 0) & (local < QR)
        osl = pl.ds(ptr, 16)
        plsc.store_compressed(pkd_v.at[osl], g, mask=mask)
        plsc.store_compressed(dst_v.at[osl], local, mask=mask)
        plsc.store_compressed(nrm_v.at[osl], nv, mask=mask)
        return ptr + plsc.all_reduce_population_count(mask)[0]

    ptr = lax.fori_loop(0, EWA // 16, filt, 0)

    for k in range(9):
        sl = pl.ds(ptr + k * 16, 16)
        pkd_v[sl] = jnp.zeros((16,), jnp.int32)
        dst_v[sl] = jnp.full((16,), QR, jnp.int32)
        nrm_v[sl] = jnp.zeros((16,), jnp.float32)
    nch = (ptr + C - 1) // C

    def to2d(j, _):
        for k in range(C // 16):
            dstc2[j, pl.ds(k * 16, 16)] = dst_v[pl.ds(j * C + k * 16, 16)]
        return 0

    lax.fori_loop(0, nch, to2d, 0)
    plsc.subcore_barrier()

    def chunk(ch, _):
        pltpu.sync_copy(table.at[pkd_v.at[pl.ds(ch * C, C)]], rows)

        def scale(g, _):
            nvec = nrm_v[pl.ds(ch * C + g * 16, 16)]
            for l in range(16):
                nv = nvec[l]
                e = g * 16 + l
                for k in range(D // 16):
                    sl = pl.ds(k * 16, 16)
                    rows[e, sl] = rows[e, sl] * nv
            return 0

        lax.fori_loop(0, C // 16, scale, 0)
        pltpu.sync_copy(rows, acc.at[dstc2.at[ch]], add=True)
        return 0

    lax.fori_loop(0, nch, chunk, 0)
    plsc.subcore_barrier()

    def flush(q, _):
        pltpu.sync_copy(acc.at[pl.ds(sid * ORT + q * FB, FB)],
                        rows.at[pl.ds(0, FB)])

        def brelu(j, _):
            for k in range(D // 16):
                sl = pl.ds(k * 16, 16)
                rows[j, sl] = jnp.maximum(rows[j, sl] + bias_v[sl], 0.0)
            return 0

        lax.fori_loop(0, FB, brelu, 0)
        pltpu.sync_copy(rows.at[pl.ds(0, FB)],
                        out.at[pl.ds(base + sid * ORT + q * FB, FB)])
        return 0

    lax.fori_loop(0, ORT // FB, flush, 0)


@functools.cache
def _sc_edge_kernel():
    return pl.kernel(
        _sc_edge_body,
        out_type=jax.ShapeDtypeStruct((NPAD, D), jnp.float32),
        mesh=plsc.VectorSubcoreMesh(core_axis_name="c", subcore_axis_name="s",
                                    num_cores=NC, num_subcores=NS),
        compiler_params=pltpu.CompilerParams(needs_layout_passes=False),
        scratch_types=[
            pltpu.VMEM((EWP,), jnp.int32),
            pltpu.VMEM((EWP,), jnp.int32),
            pltpu.VMEM((EWP,), jnp.float32),
            pltpu.VMEM((EWA // C + 2, C), jnp.int32),
            pltpu.VMEM((C, D), jnp.float32),
            pltpu.VMEM((D,), jnp.float32),
            pltpu.VMEM_SHARED((ACC_R, D), jnp.float32),
            pltpu.SemaphoreType.DMA,
        ],
    )


def _sc_edge(table, pkdr, nrmr, bias):
    return _sc_edge_kernel()(table, pkdr, nrmr, bias)


def kernel(feats, edge_index, edge_type, edge_norm,
           w_comp1, bases1, bias1, w_comp2, bases2, bias2):
    gidx = edge_type * NPAD + edge_index[0]
    packed = (edge_index[1].astype(jnp.uint32) << 18) | gidx.astype(jnp.uint32)
    packed = lax.bitcast_convert_type(packed, jnp.int32)

    def shard(a):
        return jnp.pad(a.reshape(NS, EW), ((0, 0), (0, EWA - EW)))

    pkd = shard(packed)
    nrm = shard(edge_norm.reshape(E))

    xpad = jnp.pad(feats, ((0, NPAD - N), (0, 0)))
    table1 = _tc_table(xpad, w_comp1, bases1)
    h1 = _sc_edge(table1, pkd, nrm, bias1)
    table2 = _tc_table(h1, w_comp2, bases2)
    h2 = _sc_edge(table2, pkd, nrm, bias2)
    return h2[:N]
